# baseline (speedup 1.0000x reference)
import functools

import jax
import jax.numpy as jnp
from jax import lax
from jax.experimental import pallas as pl
from jax.experimental.pallas import tpu as pltpu
from jax.experimental.pallas import tpu_sc as plsc

_N = 10000
_NP = 10240
_D = 128
_H = 32
_E = 320000
_NC = 2
_NS = 16
_NW = _NC * _NS
_EPT = _E // _NW
_K = 128
_EPW = 10240
_EPAD = _NW * _EPW - _E
_CHUNKS = _EPW // _K
_CH = _CHUNKS // 2
_RPT = _NP // _NS
_ZR = 128
_DEGW = 16

_mesh = plsc.VectorSubcoreMesh(
    core_axis_name="c", subcore_axis_name="s",
    num_cores=_NC, num_subcores=_NS)


def _deg_body(dst_hbm, out_hbm, didx, ones, zbuf, acc, sem):
    c = lax.axis_index("c")
    s = lax.axis_index("s")
    wid = c * _NS + s

    pltpu.async_copy(dst_hbm.at[wid], didx, sem)

    def _fill(i, carry):
        ones[i] = jnp.ones((_DEGW,), jnp.float32)
        return carry
    lax.fori_loop(0, _K, _fill, 0)

    def _fillz(i, carry):
        zbuf[i] = jnp.zeros((_DEGW,), jnp.float32)
        return carry
    lax.fori_loop(0, _RPT, _fillz, 0)

    pltpu.sync_copy(zbuf, acc.at[pl.ds(s * _RPT, _RPT)])
    plsc.subcore_barrier()

    pltpu.make_async_copy(dst_hbm.at[wid], didx, sem).wait()

    def _body(j, carry):
        pltpu.sync_copy(ones, acc.at[didx.at[j]], add=True)
        return carry
    lax.fori_loop(0, _CHUNKS, _body, 0)

    plsc.subcore_barrier()
    pltpu.sync_copy(acc.at[pl.ds(s * _RPT, _RPT)],
                    out_hbm.at[c, pl.ds(s * _RPT, _RPT)])


_deg_call = functools.partial(
    pl.kernel,
    out_type=jax.ShapeDtypeStruct((_NC, _NP, _DEGW), jnp.float32),
    mesh=_mesh,
    compiler_params=pltpu.CompilerParams(use_tc_tiling_on_sc=False),
    scratch_types=[
        pltpu.VMEM((_CHUNKS, _K), jnp.int32),
        pltpu.VMEM((_K, _DEGW), jnp.float32),
        pltpu.VMEM((_RPT, _DEGW), jnp.float32),
        pltpu.VMEM_SHARED((_NP, _DEGW), jnp.float32),
        pltpu.SemaphoreType.DMA,
    ],
)(_deg_body)


def _scatter_body(y_hbm, src_hbm, dst_hbm, out_hbm,
                  sidx, didx, gbuf0, gbuf1, acc, sem0, sem1):
    c = lax.axis_index("c")
    s = lax.axis_index("s")
    wid = c * _NS + s

    pltpu.sync_copy(src_hbm.at[wid, pl.ds(0, _CH)], sidx)
    pltpu.sync_copy(dst_hbm.at[wid, pl.ds(0, _CH)], didx)
    pltpu.async_copy(y_hbm.at[sidx.at[0]], gbuf0, sem0)

    def _fillz(i, carry):
        r = i // 8
        col = (i % 8) * 16
        gbuf1[r, pl.ds(col, 16)] = jnp.zeros((16,), jnp.float32)
        return carry
    lax.fori_loop(0, _K * 8, _fillz, 0)

    for i in range(_RPT // _K):
        pltpu.async_copy(gbuf1, acc.at[pl.ds(s * _RPT + i * _K, _K)], sem1)
    for i in range(_RPT // _K):
        pltpu.make_async_copy(gbuf1, acc.at[pl.ds(s * _RPT + i * _K, _K)],
                              sem1).wait()
    plsc.subcore_barrier()

    for ph in range(2):
        if ph:
            pltpu.sync_copy(src_hbm.at[wid, pl.ds(ph * _CH, _CH)], sidx)
            pltpu.sync_copy(dst_hbm.at[wid, pl.ds(ph * _CH, _CH)], didx)
            pltpu.async_copy(y_hbm.at[sidx.at[0]], gbuf0, sem0)

        def _body(i, carry):
            k = 2 * i
            pltpu.async_copy(y_hbm.at[sidx.at[k + 1]], gbuf1, sem1)
            pltpu.make_async_copy(y_hbm.at[sidx.at[k]], gbuf0, sem0).wait()
            pltpu.sync_copy(gbuf0, acc.at[didx.at[k]], add=True)

            @pl.when(k + 2 < _CH)
            def _():
                pltpu.async_copy(y_hbm.at[sidx.at[k + 2]], gbuf0, sem0)
            pltpu.make_async_copy(y_hbm.at[sidx.at[k + 1]], gbuf1, sem1).wait()
            pltpu.sync_copy(gbuf1, acc.at[didx.at[k + 1]], add=True)
            return carry
        lax.fori_loop(0, _CH // 2, _body, 0)

    plsc.subcore_barrier()
    pltpu.sync_copy(acc.at[pl.ds(s * _RPT, _RPT)],
                    out_hbm.at[c, pl.ds(s * _RPT, _RPT)])


_scatter_call = functools.partial(
    pl.kernel,
    out_type=jax.ShapeDtypeStruct((_NC, _NP, _D), jnp.float32),
    mesh=_mesh,
    scratch_types=[
        pltpu.VMEM((_CH, _K), jnp.int32),
        pltpu.VMEM((_CH, _K), jnp.int32),
        pltpu.VMEM((_K, _D), jnp.float32),
        pltpu.VMEM((_K, _D), jnp.float32),
        pltpu.VMEM_SHARED((_NP, _D), jnp.float32),
        pltpu.SemaphoreType.DMA,
        pltpu.SemaphoreType.DMA,
    ],
)(_scatter_body)


_B = 5000
_G = _N // _B


def _dinv_of(d0, d1):
    deg = d0[0, :, 0:1] + d1[0, :, 0:1] + 1.0
    return lax.rsqrt(deg)


def _mm1_body(x, w, d0, d1, y):
    dinv = _dinv_of(d0, d1)
    y[...] = jnp.dot(x[...], w[...],
                     preferred_element_type=jnp.float32) * dinv


def _fuse_body(s0, s1, y, d0, d1, b, w, out, y2):
    dinv = _dinv_of(d0, d1)
    o = jnp.maximum(dinv * (s0[0] + s1[0] + y[...]) + b[...], 0.0)
    out[...] = o
    y2[...] = jnp.dot(o, w[...], preferred_element_type=jnp.float32) * dinv


def _final_body(s0, s1, y3, d0, d1, b3, o1, o2, xs,
                l1a, l1b, l1c, l1d, lb1, lw2, lb2, lw3t, lb3,
                c_ref, ss_ref, sa_ref):
    dinv = _dinv_of(d0, d1)
    o3 = jnp.maximum(dinv * (s0[0] + s1[0] + y3[...]) + b3[...], 0.0)
    h = (jnp.dot(o1[...], l1a[...], preferred_element_type=jnp.float32)
         + jnp.dot(o2[...], l1b[...], preferred_element_type=jnp.float32)
         + jnp.dot(o3, l1c[...], preferred_element_type=jnp.float32)
         + jnp.dot(xs[...], l1d[...], preferred_element_type=jnp.float32)
         + lb1[...])
    h = jnp.where(h >= 0.0, h, 0.01 * h)
    h2 = jnp.dot(h, lw2[...], preferred_element_type=jnp.float32) + lb2[...]
    h2 = jnp.where(h2 >= 0.0, h2, 0.01 * h2)
    z = jnp.sum(h2 * lw3t[...], axis=1, keepdims=True) + lb3[...]
    cc = jnp.maximum(z, 0.0) + jnp.log1p(jnp.exp(-jnp.abs(z)))
    c_ref[...] = cc

    @pl.when(pl.program_id(0) == 0)
    def _():
        ss_ref[...] = jnp.zeros((1, 1), jnp.float32)
        sa_ref[...] = jnp.zeros((1, 1), jnp.float32)
    ss_ref[...] += jnp.sum(cc).reshape(1, 1)
    sa_ref[...] += jnp.sum(jnp.abs(cc)).reshape(1, 1)


def _row_spec(w):
    return pl.BlockSpec((_B, w), lambda i: (i, 0))


def _part_spec(p, w):
    return pl.BlockSpec((1, _B, w), lambda i, _p=p: (_p, i, 0))


def _full_spec(shape):
    return pl.BlockSpec(shape, lambda i: tuple(0 for _ in shape))


_mm1 = pl.pallas_call(
    _mm1_body,
    grid=(_G,),
    in_specs=[_row_spec(_D), _full_spec((_D, _D)),
              _part_spec(0, _DEGW), _part_spec(1, _DEGW)],
    out_specs=_row_spec(_D),
    out_shape=jax.ShapeDtypeStruct((_N, _D), jnp.float32),
)

_fuse = pl.pallas_call(
    _fuse_body,
    grid=(_G,),
    in_specs=[_part_spec(0, _D), _part_spec(1, _D), _row_spec(_D),
              _part_spec(0, _DEGW), _part_spec(1, _DEGW),
              _full_spec((1, _D)), _full_spec((_D, _D))],
    out_specs=[_row_spec(_D), _row_spec(_D)],
    out_shape=[jax.ShapeDtypeStruct((_N, _D), jnp.float32),
               jax.ShapeDtypeStruct((_N, _D), jnp.float32)],
)

_final = pl.pallas_call(
    _final_body,
    grid=(_G,),
    in_specs=[_part_spec(0, _D), _part_spec(1, _D), _row_spec(_D),
              _part_spec(0, _DEGW), _part_spec(1, _DEGW),
              _full_spec((1, _D)),
              _row_spec(_D), _row_spec(_D), _row_spec(_D),
              _full_spec((_D, _H)), _full_spec((_D, _H)),
              _full_spec((_D, _H)), _full_spec((_D, _H)),
              _full_spec((1, _H)), _full_spec((_H, _H)),
              _full_spec((1, _H)), _full_spec((1, _H)),
              _full_spec((1, 1))],
    out_specs=[_row_spec(1),
               pl.BlockSpec((1, 1), lambda i: (0, 0)),
               pl.BlockSpec((1, 1), lambda i: (0, 0))],
    out_shape=[jax.ShapeDtypeStruct((_N, 1), jnp.float32),
               jax.ShapeDtypeStruct((1, 1), jnp.float32),
               jax.ShapeDtypeStruct((1, 1), jnp.float32)],
)


def kernel(state, edge_index, W1, b1, W2, b2, W3, b3,
           lw1, lb1, lw2, lb2, lw3, lb3):
    pidx = jnp.arange(_EPAD, dtype=jnp.int32)
    psrc = pidx % _N
    pdst = _N + pidx % (_NP - _N)
    src3 = jnp.concatenate([edge_index[0], psrc]).reshape(_NW, _CHUNKS, _K)
    dst3 = jnp.concatenate([edge_index[1], pdst]).reshape(_NW, _CHUNKS, _K)

    degp = _deg_call(dst3)

    y1 = _mm1(state, W1, degp, degp)
    S1 = _scatter_call(y1, src3, dst3)
    out1, y2 = _fuse(S1, S1, y1, degp, degp, b1.reshape(1, _D), W2)
    S2 = _scatter_call(y2, src3, dst3)
    out2, y3 = _fuse(S2, S2, y2, degp, degp, b2.reshape(1, _D), W3)
    S3 = _scatter_call(y3, src3, dst3)

    conc, ssum, sabs = _final(
        S3, S3, y3, degp, degp, b3.reshape(1, _D),
        out1, out2, state,
        lw1[0:_D], lw1[_D:2 * _D], lw1[2 * _D:3 * _D], lw1[3 * _D:4 * _D],
        lb1.reshape(1, _H), lw2, lb2.reshape(1, _H),
        lw3.reshape(1, _H), lb3.reshape(1, 1))

    c = conc[:, 0]
    action = c / (ssum[0, 0] + 1e-20)
    regularize = sabs[0, 0] / _N
    return action, regularize


# speedup vs baseline: 1.0097x; 1.0097x over previous
import functools

import jax
import jax.numpy as jnp
from jax import lax
from jax.experimental import pallas as pl
from jax.experimental.pallas import tpu as pltpu
from jax.experimental.pallas import tpu_sc as plsc

_N = 10000
_NP = 10240
_D = 128
_H = 32
_E = 320000
_NC = 2
_NS = 16
_NW = _NC * _NS
_EPT = _E // _NW
_K = 128
_EPW = 10240
_EPAD = _NW * _EPW - _E
_CHUNKS = _EPW // _K
_CH = _CHUNKS // 2
_RPT = _NP // _NS
_ZR = 128
_DEGW = 16

_mesh = plsc.VectorSubcoreMesh(
    core_axis_name="c", subcore_axis_name="s",
    num_cores=_NC, num_subcores=_NS)


def _deg_body(dst_hbm, out_hbm, didx, ones, zbuf, acc, sem):
    c = lax.axis_index("c")
    s = lax.axis_index("s")
    wid = c * _NS + s

    pltpu.async_copy(dst_hbm.at[wid], didx, sem)

    def _fill(i, carry):
        ones[i] = jnp.ones((_DEGW,), jnp.float32)
        return carry
    lax.fori_loop(0, _K, _fill, 0)

    def _fillz(i, carry):
        zbuf[i] = jnp.zeros((_DEGW,), jnp.float32)
        return carry
    lax.fori_loop(0, _RPT, _fillz, 0)

    pltpu.sync_copy(zbuf, acc.at[pl.ds(s * _RPT, _RPT)])
    plsc.subcore_barrier()

    pltpu.make_async_copy(dst_hbm.at[wid], didx, sem).wait()

    def _body(j, carry):
        pltpu.sync_copy(ones, acc.at[didx.at[j]], add=True)
        return carry
    lax.fori_loop(0, _CHUNKS, _body, 0)

    plsc.subcore_barrier()
    pltpu.sync_copy(acc.at[pl.ds(s * _RPT, _RPT)],
                    out_hbm.at[c, pl.ds(s * _RPT, _RPT)])


_deg_call = functools.partial(
    pl.kernel,
    out_type=jax.ShapeDtypeStruct((_NC, _NP, _DEGW), jnp.float32),
    mesh=_mesh,
    compiler_params=pltpu.CompilerParams(use_tc_tiling_on_sc=False),
    scratch_types=[
        pltpu.VMEM((_CHUNKS, _K), jnp.int32),
        pltpu.VMEM((_K, _DEGW), jnp.float32),
        pltpu.VMEM((_RPT, _DEGW), jnp.float32),
        pltpu.VMEM_SHARED((_NP, _DEGW), jnp.float32),
        pltpu.SemaphoreType.DMA,
    ],
)(_deg_body)


def _scatter_body(y_hbm, src_hbm, dst_hbm, out_hbm,
                  sidx, didx, gbuf0, gbuf1, acc, sem0, sem1):
    c = lax.axis_index("c")
    s = lax.axis_index("s")
    wid = c * _NS + s

    pltpu.sync_copy(src_hbm.at[wid, pl.ds(0, _CH)], sidx)
    pltpu.sync_copy(dst_hbm.at[wid, pl.ds(0, _CH)], didx)
    pltpu.async_copy(y_hbm.at[sidx.at[0]], gbuf0, sem0)

    def _fillz(i, carry):
        r = i // 8
        col = (i % 8) * 16
        gbuf1[r, pl.ds(col, 16)] = jnp.zeros((16,), jnp.float32)
        return carry
    lax.fori_loop(0, _K * 8, _fillz, 0)

    for i in range(_RPT // _K):
        pltpu.async_copy(gbuf1, acc.at[pl.ds(s * _RPT + i * _K, _K)], sem1)
    for i in range(_RPT // _K):
        pltpu.make_async_copy(gbuf1, acc.at[pl.ds(s * _RPT + i * _K, _K)],
                              sem1).wait()
    plsc.subcore_barrier()

    for ph in range(2):
        if ph:
            pltpu.sync_copy(src_hbm.at[wid, pl.ds(ph * _CH, _CH)], sidx)
            pltpu.sync_copy(dst_hbm.at[wid, pl.ds(ph * _CH, _CH)], didx)
            pltpu.async_copy(y_hbm.at[sidx.at[0]], gbuf0, sem0)

        def _body(i, carry):
            k = 2 * i
            pltpu.async_copy(y_hbm.at[sidx.at[k + 1]], gbuf1, sem1)
            pltpu.make_async_copy(y_hbm.at[sidx.at[k]], gbuf0, sem0).wait()
            pltpu.sync_copy(gbuf0, acc.at[didx.at[k]], add=True)

            @pl.when(k + 2 < _CH)
            def _():
                pltpu.async_copy(y_hbm.at[sidx.at[k + 2]], gbuf0, sem0)
            pltpu.make_async_copy(y_hbm.at[sidx.at[k + 1]], gbuf1, sem1).wait()
            pltpu.sync_copy(gbuf1, acc.at[didx.at[k + 1]], add=True)
            return carry
        lax.fori_loop(0, _CH // 2, _body, 0)

    plsc.subcore_barrier()
    pltpu.sync_copy(acc.at[pl.ds(s * _RPT, _RPT)],
                    out_hbm.at[c, pl.ds(s * _RPT, _RPT)])


_scatter_call = functools.partial(
    pl.kernel,
    out_type=jax.ShapeDtypeStruct((_NC, _NP, _D), jnp.float32),
    mesh=_mesh,
    scratch_types=[
        pltpu.VMEM((_CH, _K), jnp.int32),
        pltpu.VMEM((_CH, _K), jnp.int32),
        pltpu.VMEM((_K, _D), jnp.float32),
        pltpu.VMEM((_K, _D), jnp.float32),
        pltpu.VMEM_SHARED((_NP, _D), jnp.float32),
        pltpu.SemaphoreType.DMA,
        pltpu.SemaphoreType.DMA,
    ],
)(_scatter_body)


_B = 2000
_G = _N // _B


def _dinv_of(d0, d1):
    deg = d0[0, :, 0:1] + d1[0, :, 0:1] + 1.0
    return lax.rsqrt(deg)


def _mm1_body(x, w, d0, d1, y):
    dinv = _dinv_of(d0, d1)
    y[...] = jnp.dot(x[...], w[...],
                     preferred_element_type=jnp.float32) * dinv


def _fuse_body(s0, s1, y, d0, d1, b, w, out, y2):
    dinv = _dinv_of(d0, d1)
    o = jnp.maximum(dinv * (s0[0] + s1[0] + y[...]) + b[...], 0.0)
    out[...] = o
    y2[...] = jnp.dot(o, w[...], preferred_element_type=jnp.float32) * dinv


def _final_body(s0, s1, y3, d0, d1, b3, o1, o2, xs,
                l1a, l1b, l1c, l1d, lb1, lw2, lb2, lw3t, lb3,
                c_ref, ss_ref, sa_ref):
    dinv = _dinv_of(d0, d1)
    o3 = jnp.maximum(dinv * (s0[0] + s1[0] + y3[...]) + b3[...], 0.0)
    h = (jnp.dot(o1[...], l1a[...], preferred_element_type=jnp.float32)
         + jnp.dot(o2[...], l1b[...], preferred_element_type=jnp.float32)
         + jnp.dot(o3, l1c[...], preferred_element_type=jnp.float32)
         + jnp.dot(xs[...], l1d[...], preferred_element_type=jnp.float32)
         + lb1[...])
    h = jnp.where(h >= 0.0, h, 0.01 * h)
    h2 = jnp.dot(h, lw2[...], preferred_element_type=jnp.float32) + lb2[...]
    h2 = jnp.where(h2 >= 0.0, h2, 0.01 * h2)
    z = jnp.sum(h2 * lw3t[...], axis=1, keepdims=True) + lb3[...]
    cc = jnp.maximum(z, 0.0) + jnp.log1p(jnp.exp(-jnp.abs(z)))
    c_ref[...] = cc

    @pl.when(pl.program_id(0) == 0)
    def _():
        ss_ref[...] = jnp.zeros((1, 1), jnp.float32)
        sa_ref[...] = jnp.zeros((1, 1), jnp.float32)
    ss_ref[...] += jnp.sum(cc).reshape(1, 1)
    sa_ref[...] += jnp.sum(jnp.abs(cc)).reshape(1, 1)


def _row_spec(w):
    return pl.BlockSpec((_B, w), lambda i: (i, 0))


def _part_spec(p, w):
    return pl.BlockSpec((1, _B, w), lambda i, _p=p: (_p, i, 0))


def _full_spec(shape):
    return pl.BlockSpec(shape, lambda i: tuple(0 for _ in shape))


_mm1 = pl.pallas_call(
    _mm1_body,
    grid=(_G,),
    in_specs=[_row_spec(_D), _full_spec((_D, _D)),
              _part_spec(0, _DEGW), _part_spec(1, _DEGW)],
    out_specs=_row_spec(_D),
    out_shape=jax.ShapeDtypeStruct((_N, _D), jnp.float32),
)

_fuse = pl.pallas_call(
    _fuse_body,
    grid=(_G,),
    in_specs=[_part_spec(0, _D), _part_spec(1, _D), _row_spec(_D),
              _part_spec(0, _DEGW), _part_spec(1, _DEGW),
              _full_spec((1, _D)), _full_spec((_D, _D))],
    out_specs=[_row_spec(_D), _row_spec(_D)],
    out_shape=[jax.ShapeDtypeStruct((_N, _D), jnp.float32),
               jax.ShapeDtypeStruct((_N, _D), jnp.float32)],
)

_final = pl.pallas_call(
    _final_body,
    grid=(_G,),
    in_specs=[_part_spec(0, _D), _part_spec(1, _D), _row_spec(_D),
              _part_spec(0, _DEGW), _part_spec(1, _DEGW),
              _full_spec((1, _D)),
              _row_spec(_D), _row_spec(_D), _row_spec(_D),
              _full_spec((_D, _H)), _full_spec((_D, _H)),
              _full_spec((_D, _H)), _full_spec((_D, _H)),
              _full_spec((1, _H)), _full_spec((_H, _H)),
              _full_spec((1, _H)), _full_spec((1, _H)),
              _full_spec((1, 1))],
    out_specs=[_row_spec(1),
               pl.BlockSpec((1, 1), lambda i: (0, 0)),
               pl.BlockSpec((1, 1), lambda i: (0, 0))],
    out_shape=[jax.ShapeDtypeStruct((_N, 1), jnp.float32),
               jax.ShapeDtypeStruct((1, 1), jnp.float32),
               jax.ShapeDtypeStruct((1, 1), jnp.float32)],
)


def kernel(state, edge_index, W1, b1, W2, b2, W3, b3,
           lw1, lb1, lw2, lb2, lw3, lb3):
    pidx = jnp.arange(_EPAD, dtype=jnp.int32)
    psrc = pidx % _N
    pdst = _N + pidx % (_NP - _N)
    src3 = jnp.concatenate([edge_index[0], psrc]).reshape(_NW, _CHUNKS, _K)
    dst3 = jnp.concatenate([edge_index[1], pdst]).reshape(_NW, _CHUNKS, _K)

    degp = _deg_call(dst3)

    y1 = _mm1(state, W1, degp, degp)
    S1 = _scatter_call(y1, src3, dst3)
    out1, y2 = _fuse(S1, S1, y1, degp, degp, b1.reshape(1, _D), W2)
    S2 = _scatter_call(y2, src3, dst3)
    out2, y3 = _fuse(S2, S2, y2, degp, degp, b2.reshape(1, _D), W3)
    S3 = _scatter_call(y3, src3, dst3)

    conc, ssum, sabs = _final(
        S3, S3, y3, degp, degp, b3.reshape(1, _D),
        out1, out2, state,
        lw1[0:_D], lw1[_D:2 * _D], lw1[2 * _D:3 * _D], lw1[3 * _D:4 * _D],
        lb1.reshape(1, _H), lw2, lb2.reshape(1, _H),
        lw3.reshape(1, _H), lb3.reshape(1, 1))

    c = conc[:, 0]
    action = c / (ssum[0, 0] + 1e-20)
    regularize = sabs[0, 0] / _N
    return action, regularize
